# baseline (speedup 1.0000x reference)
import functools

import jax
import jax.numpy as jnp
from jax import lax
from jax.experimental import pallas as pl
from jax.experimental.pallas import tpu as pltpu
from jax.experimental.pallas import tpu_sc as plsc

N = 10000
NP1 = 10240
E = 320000
D = 128
K = 4
NC, NS = 2, 16
NW = NC * NS
CHUNK = 128
NCH = 81
EPW = CHUNK * NCH
EPAD = EPW * NW

ROWS_BLK = 1000

_mesh = plsc.VectorSubcoreMesh(core_axis_name="c", subcore_axis_name="s")


def _strip_copies(sid, fn):
    base = sid * 632
    for off in (0, 128, 256, 384):
        fn(base + off, 128)

    @pl.when(sid < NS - 1)
    def _full():
        fn(base + 512, 120)

    @pl.when(sid == NS - 1)
    def _last():
        fn(base + 512, 8)


def _zero16():
    return jnp.zeros((16,), jnp.float32)


@functools.partial(
    pl.kernel,
    mesh=_mesh,
    compiler_params=pltpu.CompilerParams(needs_layout_passes=False),
    out_type=jax.ShapeDtypeStruct((NC, NP1), jnp.float32),
    scratch_types=[
        pltpu.VMEM((NCH, CHUNK), jnp.int32),
        pltpu.VMEM((NCH, CHUNK), jnp.float32),
        pltpu.VMEM((NP1 // NS,), jnp.float32),
        pltpu.VMEM_SHARED((NP1,), jnp.float32),
    ],
)
def _deg_kernel(row_hbm, w_hbm, out_hbm, row_v, w_v, zbuf, accum):
    cid = lax.axis_index("c")
    sid = lax.axis_index("s")
    wid = cid * NS + sid
    strip = NP1 // NS

    def zb(i, carry):
        zbuf[pl.ds(i * 16, 16)] = _zero16()
        return carry

    lax.fori_loop(0, strip // 16, zb, 0)
    pltpu.sync_copy(zbuf, accum.at[pl.ds(sid * strip, strip)])
    pltpu.sync_copy(row_hbm.at[wid], row_v)
    pltpu.sync_copy(w_hbm.at[wid], w_v)
    plsc.subcore_barrier()

    def body(j, carry):
        pltpu.sync_copy(w_v.at[j], accum.at[row_v.at[j]], add=True)
        return carry

    lax.fori_loop(0, NCH, body, 0)
    plsc.subcore_barrier()
    pltpu.sync_copy(accum.at[pl.ds(sid * strip, strip)],
                    out_hbm.at[cid, pl.ds(sid * strip, strip)])


@functools.partial(
    pl.kernel,
    mesh=_mesh,
    compiler_params=pltpu.CompilerParams(needs_layout_passes=False),
    out_type=jax.ShapeDtypeStruct((NW, NCH, CHUNK), jnp.float32),
    scratch_types=[
        pltpu.VMEM((NP1,), jnp.float32),
        pltpu.VMEM((NP1,), jnp.float32),
        pltpu.VMEM((NP1,), jnp.float32),
        pltpu.VMEM((NCH, CHUNK), jnp.int32),
        pltpu.VMEM((NCH, CHUNK), jnp.int32),
        pltpu.VMEM((NCH, CHUNK), jnp.float32),
        pltpu.VMEM((NCH, CHUNK), jnp.float32),
    ],
)
def _norm_kernel(degp_hbm, row_hbm, col_hbm, w_hbm, norm_hbm,
                 d0_v, d1_v, dis_v, row_v, col_v, w_v, norm_v):
    cid = lax.axis_index("c")
    sid = lax.axis_index("s")
    wid = cid * NS + sid
    pltpu.sync_copy(degp_hbm.at[0], d0_v)
    pltpu.sync_copy(degp_hbm.at[1], d1_v)
    pltpu.sync_copy(row_hbm.at[wid], row_v)
    pltpu.sync_copy(col_hbm.at[wid], col_v)
    pltpu.sync_copy(w_hbm.at[wid], w_v)

    magic = jnp.full((16,), 0x5F3759DF, jnp.int32)

    def dbody(i, carry):
        sl = pl.ds(i * 16, 16)
        d = d0_v[sl] + d1_v[sl]
        bits = lax.bitcast_convert_type(d, jnp.int32)
        y = lax.bitcast_convert_type(
            magic - lax.shift_right_logical(bits, 1), jnp.float32)
        y = y * (1.5 - 0.5 * d * y * y)
        y = y * (1.5 - 0.5 * d * y * y)
        y = y * (1.5 - 0.5 * d * y * y)
        dis_v[sl] = jnp.where(d > 0.0, y, 0.0)
        return carry

    lax.fori_loop(0, NP1 // 16, dbody, 0)

    def nbody(j, carry):
        def inner(m, carry2):
            sl = pl.ds(m * 16, 16)
            rv = row_v[j, sl]
            cv = col_v[j, sl]
            wv = w_v[j, sl]
            disr = plsc.load_gather(dis_v, [rv])
            disc = plsc.load_gather(dis_v, [cv])
            nv = -disr * wv * disc - jnp.where(rv == cv, 1.0, 0.0)
            norm_v[j, sl] = nv
            return carry2

        return lax.fori_loop(0, CHUNK // 16, inner, carry)

    lax.fori_loop(0, NCH, nbody, 0)
    pltpu.sync_copy(norm_v, norm_hbm.at[wid])


@functools.partial(
    pl.kernel,
    mesh=_mesh,
    compiler_params=pltpu.CompilerParams(needs_layout_passes=False),
    out_type=jax.ShapeDtypeStruct((NC, N, D), jnp.float32),
    scratch_types=[
        pltpu.VMEM((3, CHUNK), jnp.int32),
        pltpu.VMEM((3, CHUNK), jnp.int32),
        pltpu.VMEM((3, CHUNK), jnp.float32),
        pltpu.VMEM((CHUNK, D), jnp.float32),
        pltpu.VMEM((CHUNK, D), jnp.float32),
        pltpu.VMEM((CHUNK, D), jnp.float32),
        pltpu.VMEM_SHARED((N, D), jnp.float32),
        pltpu.SemaphoreType.DMA,
        pltpu.SemaphoreType.DMA,
        pltpu.SemaphoreType.DMA,
        pltpu.SemaphoreType.DMA,
        pltpu.SemaphoreType.DMA,
        pltpu.SemaphoreType.DMA,
        pltpu.SemaphoreType.DMA,
        pltpu.SemaphoreType.DMA,
        pltpu.SemaphoreType.DMA,
        pltpu.SemaphoreType.DMA,
        pltpu.SemaphoreType.DMA,
        pltpu.SemaphoreType.DMA,
        pltpu.SemaphoreType.DMA,
        pltpu.SemaphoreType.DMA,
        pltpu.SemaphoreType.DMA,
    ],
)
def _prop_kernel(z_hbm, row_hbm, col_hbm, norm_hbm, out_hbm,
                 rowr, colr, nrmr, gb0, gb1, gb2, accum,
                 gs0, gs1, gs2, ss0, ss1, ss2,
                 rs0, rs1, rs2, cs0, cs1, cs2, ns0, ns1, ns2):
    cid = lax.axis_index("c")
    sid = lax.axis_index("s")
    wid = cid * NS + sid
    gbufs = (gb0, gb1, gb2)
    gsems = (gs0, gs1, gs2)
    ssems = (ss0, ss1, ss2)
    rsems = (rs0, rs1, rs2)
    csems = (cs0, cs1, cs2)
    nsems = (ns0, ns1, ns2)

    def zb(r, carry):
        for m in range(D // 16):
            gb0[r, pl.ds(m * 16, 16)] = _zero16()
        return carry

    lax.fori_loop(0, CHUNK, zb, 0)
    _strip_copies(sid, lambda off, sz: pltpu.sync_copy(
        gb0.at[pl.ds(0, sz)], accum.at[pl.ds(off, sz)]))
    plsc.subcore_barrier()

    def row_dma(j, b):
        return pltpu.make_async_copy(
            row_hbm.at[pl.ds(wid * EPW + j * CHUNK, CHUNK)], rowr.at[b],
            rsems[b])

    def col_dma(j, b):
        return pltpu.make_async_copy(
            col_hbm.at[pl.ds(wid * EPW + j * CHUNK, CHUNK)], colr.at[b],
            csems[b])

    def nrm_dma(j, b):
        return pltpu.make_async_copy(
            norm_hbm.at[pl.ds(wid * EPW + j * CHUNK, CHUNK)], nrmr.at[b],
            nsems[b])

    def gather_dma(b):
        return pltpu.make_async_copy(z_hbm.at[rowr.at[b]], gbufs[b],
                                     gsems[b])

    def scatter_dma(b):
        return pltpu.make_async_copy(gbufs[b], accum.at[colr.at[b]],
                                     ssems[b])

    def scale(b, buf):
        @plsc.parallel_loop(0, CHUNK, unroll=8)
        def _(r):
            nv = plsc.load_gather(nrmr.at[b],
                                  [jnp.full((16,), r, jnp.int32)])
            for m in range(D // 16):
                sl = pl.ds(m * 16, 16)
                buf[r, sl] = buf[r, sl] * nv

    for jj in range(3):
        row_dma(jj, jj).start()
        nrm_dma(jj, jj).start()
    for jj in range(2):
        col_dma(jj, jj).start()
    for b in range(2):
        row_dma(b, b).wait()
        gather_dma(b).start()

    def triple(t, carry):
        for p in range(3):
            j = 3 * t + p
            ba, bc = p % 3, (p + 2) % 3
            last = NCH // 3 - 1

            def prefetch():
                row_dma(j + 2, bc).wait()
                gather_dma(bc).start()
                col_dma(j + 2, bc).start()

            if p == 0:
                @pl.when(t > 0)
                def _w():
                    scatter_dma(bc).wait()

                prefetch()
            else:
                @pl.when(t < last)
                def _n():
                    scatter_dma(bc).wait()
                    prefetch()

                @pl.when(t == last)
                def _l():
                    scatter_dma(bc).wait()

            pltpu.make_async_copy(z_hbm.at[rowr.at[ba]], gbufs[ba],
                                  gsems[ba]).wait()

            @pl.when(t < last)
            def _rnext():
                row_dma(j + 3, ba).start()

            nrm_dma(j, ba).wait()
            scale(ba, gbufs[ba])
            col_dma(j, ba).wait()
            pltpu.async_copy(gbufs[ba], accum.at[colr.at[ba]], ssems[ba],
                             add=True)

            @pl.when(t < last)
            def _nnext():
                nrm_dma(j + 3, ba).start()
        return carry

    lax.fori_loop(0, NCH // 3, triple, 0)
    scatter_dma((NCH - 1) % 3).wait()
    plsc.subcore_barrier()
    _strip_copies(sid, lambda off, sz: pltpu.sync_copy(
        accum.at[pl.ds(off, sz)], out_hbm.at[cid, pl.ds(off, sz)]))


def _comb1_body(px_ref, ph_ref, tx_ref, th_ref):
    tx_ref[...] = px_ref[0] + px_ref[1]
    th_ref[...] = ph_ref[0] + ph_ref[1]


def _comb2_body(px_ref, ph_ref, xp_ref, hp_ref, tx_ref, th_ref):
    tx_ref[...] = 2.0 * (px_ref[0] + px_ref[1]) - xp_ref[...]
    th_ref[...] = 2.0 * (ph_ref[0] + ph_ref[1]) - hp_ref[...]


def _pblk():
    return pl.BlockSpec((NC, ROWS_BLK, D), lambda i: (0, i, 0))


def _blk():
    return pl.BlockSpec((ROWS_BLK, D), lambda i: (i, 0))


def _comb1(px, ph):
    return pl.pallas_call(
        _comb1_body,
        grid=(N // ROWS_BLK,),
        in_specs=[_pblk(), _pblk()],
        out_specs=[_blk(), _blk()],
        out_shape=[jax.ShapeDtypeStruct((N, D), jnp.float32)] * 2,
    )(px, ph)


def _comb2(px, ph, xp, hp):
    return pl.pallas_call(
        _comb2_body,
        grid=(N // ROWS_BLK,),
        in_specs=[_pblk(), _pblk(), _blk(), _blk()],
        out_specs=[_blk(), _blk()],
        out_shape=[jax.ShapeDtypeStruct((N, D), jnp.float32)] * 2,
    )(px, ph, xp, hp)


def _gate_kernel(x_ref, tx1_ref, tx2_ref, px3_ref,
                 h_ref, th1_ref, th2_ref, ph3_ref,
                 c_ref, wx_ref, wh_ref, wc_ref, bias_ref,
                 lin_w_ref, lin_b_ref,
                 out_ref, h0_ref, cn_ref):
    f32 = jnp.float32

    def dot(a, b):
        return jnp.dot(a, b, preferred_element_type=f32)

    tx3 = 2.0 * (px3_ref[0] + px3_ref[1]) - tx1_ref[...]
    th3 = 2.0 * (ph3_ref[0] + ph3_ref[1]) - th1_ref[...]
    pre = (dot(x_ref[...], wx_ref[0:128])
           + dot(tx1_ref[...], wx_ref[128:256])
           + dot(tx2_ref[...], wx_ref[256:384])
           + dot(tx3, wx_ref[384:512])
           + dot(h_ref[...], wh_ref[0:128])
           + dot(th1_ref[...], wh_ref[128:256])
           + dot(th2_ref[...], wh_ref[256:384])
           + dot(th3, wh_ref[384:512])
           + bias_ref[0:1, :])
    c = c_ref[...]
    ig = jax.nn.sigmoid(pre[:, 0:128] + wc_ref[0:1, :] * c)
    fg = jax.nn.sigmoid(pre[:, 128:256] + wc_ref[1:2, :] * c)
    tg = jnp.tanh(pre[:, 256:384])
    cn = fg * c + ig * tg
    og = jax.nn.sigmoid(pre[:, 384:512] + wc_ref[2:3, :] * cn)
    h0 = og * jnp.tanh(cn)
    out = dot(jax.nn.relu(h0), lin_w_ref[...]) + lin_b_ref[0:1, :]
    out_ref[...] = out
    h0_ref[...] = h0
    cn_ref[...] = cn


def _gates(x, tx1, tx2, px3, h, th1, th2, ph3, c, wx, wh, wc, bias, lin_w, lin_b):
    full = lambda shp: pl.BlockSpec(shp, lambda i: (0,) * len(shp))
    return pl.pallas_call(
        _gate_kernel,
        grid=(N // ROWS_BLK,),
        in_specs=[
            _blk(), _blk(), _blk(), _pblk(),
            _blk(), _blk(), _blk(), _pblk(),
            _blk(),
            full((512, 512)), full((512, 512)),
            full((8, D)), full((8, 512)),
            full((D, D)), full((8, D)),
        ],
        out_specs=[_blk(), _blk(), _blk()],
        out_shape=[jax.ShapeDtypeStruct((N, D), jnp.float32)] * 3,
    )(x, tx1, tx2, px3, h, th1, th2, ph3, c, wx, wh, wc, bias, lin_w, lin_b)


def kernel(x, edge_index, edge_weight, h, c, params):
    p = params
    row = edge_index[0]
    col = edge_index[1]

    pad = EPAD - E
    apad = jnp.arange(pad, dtype=jnp.int32)
    prow = (2 * apad) % N
    pcol = (2 * apad + 1) % N
    row_p = jnp.concatenate([row, prow]).reshape(NW, NCH, CHUNK)
    col_p = jnp.concatenate([col, pcol]).reshape(NW, NCH, CHUNK)
    w_p = jnp.concatenate(
        [edge_weight, jnp.zeros((pad,), jnp.float32)]).reshape(NW, NCH, CHUNK)

    degp = _deg_kernel(row_p, w_p)
    norm = _norm_kernel(degp, row_p, col_p, w_p)

    row_f = row_p.reshape(NW * EPW)
    col_f = col_p.reshape(NW * EPW)
    norm_f = norm.reshape(NW * EPW)
    px1 = _prop_kernel(x, row_f, col_f, norm_f)
    ph1 = _prop_kernel(h, row_f, col_f, norm_f)
    tx1, th1 = _comb1(px1, ph1)
    px2 = _prop_kernel(tx1, row_f, col_f, norm_f)
    ph2 = _prop_kernel(th1, row_f, col_f, norm_f)
    tx2, th2 = _comb2(px2, ph2, x, h)
    px3 = _prop_kernel(tx2, row_f, col_f, norm_f)
    ph3 = _prop_kernel(th2, row_f, col_f, norm_f)

    gates = ["i", "f", "c", "o"]
    wx = jnp.concatenate([p["Wx_" + g] for g in gates], axis=2).reshape(K * D, 4 * D)
    wh = jnp.concatenate([p["Wh_" + g] for g in gates], axis=2).reshape(K * D, 4 * D)
    wc = jnp.concatenate(
        [p["w_c_i"], p["w_c_f"], p["w_c_o"], jnp.zeros((5, D), jnp.float32)], axis=0)
    bias = jnp.concatenate(
        [(p["bx_" + g] + p["bh_" + g])[None, :] + p["b_" + g] for g in gates], axis=1)
    bias = jnp.concatenate([bias, jnp.zeros((7, 4 * D), jnp.float32)], axis=0)
    lin_b = jnp.concatenate([p["lin_b"][None, :], jnp.zeros((7, D), jnp.float32)], axis=0)

    out, h0, cn = _gates(x, tx1, tx2, px3, h, th1, th2, ph3, c,
                         wx, wh, wc, bias, p["lin_W"], lin_b)
    return (out, h0, cn)


# speedup vs baseline: 1.0063x; 1.0063x over previous
import functools

import jax
import jax.numpy as jnp
from jax import lax
from jax.experimental import pallas as pl
from jax.experimental.pallas import tpu as pltpu
from jax.experimental.pallas import tpu_sc as plsc

N = 10000
NP1 = 10240
E = 320000
D = 128
K = 4
NC, NS = 2, 16
NW = NC * NS
CHUNK = 128
NCH = 81
EPW = CHUNK * NCH
EPAD = EPW * NW

ROWS_BLK = 1000

_mesh = plsc.VectorSubcoreMesh(core_axis_name="c", subcore_axis_name="s")


def _strip_copies(sid, fn):
    base = sid * 632
    for off in (0, 128, 256, 384):
        fn(base + off, 128)

    @pl.when(sid < NS - 1)
    def _full():
        fn(base + 512, 120)

    @pl.when(sid == NS - 1)
    def _last():
        fn(base + 512, 8)


def _zero16():
    return jnp.zeros((16,), jnp.float32)


@functools.partial(
    pl.kernel,
    mesh=_mesh,
    compiler_params=pltpu.CompilerParams(needs_layout_passes=False),
    out_type=jax.ShapeDtypeStruct((NC, NP1), jnp.float32),
    scratch_types=[
        pltpu.VMEM((NCH, CHUNK), jnp.int32),
        pltpu.VMEM((NCH, CHUNK), jnp.float32),
        pltpu.VMEM((NP1 // NS,), jnp.float32),
        pltpu.VMEM_SHARED((NP1,), jnp.float32),
    ],
)
def _deg_kernel(row_hbm, w_hbm, out_hbm, row_v, w_v, zbuf, accum):
    cid = lax.axis_index("c")
    sid = lax.axis_index("s")
    wid = cid * NS + sid
    strip = NP1 // NS

    def zb(i, carry):
        zbuf[pl.ds(i * 16, 16)] = _zero16()
        return carry

    lax.fori_loop(0, strip // 16, zb, 0)
    pltpu.sync_copy(zbuf, accum.at[pl.ds(sid * strip, strip)])
    pltpu.sync_copy(row_hbm.at[wid], row_v)
    pltpu.sync_copy(w_hbm.at[wid], w_v)
    plsc.subcore_barrier()

    def body(j, carry):
        pltpu.sync_copy(w_v.at[j], accum.at[row_v.at[j]], add=True)
        return carry

    lax.fori_loop(0, NCH, body, 0)
    plsc.subcore_barrier()
    pltpu.sync_copy(accum.at[pl.ds(sid * strip, strip)],
                    out_hbm.at[cid, pl.ds(sid * strip, strip)])


@functools.partial(
    pl.kernel,
    mesh=_mesh,
    compiler_params=pltpu.CompilerParams(needs_layout_passes=False),
    out_type=jax.ShapeDtypeStruct((NW, NCH, CHUNK), jnp.float32),
    scratch_types=[
        pltpu.VMEM((NP1,), jnp.float32),
        pltpu.VMEM((NP1,), jnp.float32),
        pltpu.VMEM((NP1,), jnp.float32),
        pltpu.VMEM((NCH, CHUNK), jnp.int32),
        pltpu.VMEM((NCH, CHUNK), jnp.int32),
        pltpu.VMEM((NCH, CHUNK), jnp.float32),
        pltpu.VMEM((NCH, CHUNK), jnp.float32),
    ],
)
def _norm_kernel(degp_hbm, row_hbm, col_hbm, w_hbm, norm_hbm,
                 d0_v, d1_v, dis_v, row_v, col_v, w_v, norm_v):
    cid = lax.axis_index("c")
    sid = lax.axis_index("s")
    wid = cid * NS + sid
    pltpu.sync_copy(degp_hbm.at[0], d0_v)
    pltpu.sync_copy(degp_hbm.at[1], d1_v)
    pltpu.sync_copy(row_hbm.at[wid], row_v)
    pltpu.sync_copy(col_hbm.at[wid], col_v)
    pltpu.sync_copy(w_hbm.at[wid], w_v)

    magic = jnp.full((16,), 0x5F3759DF, jnp.int32)

    def dbody(i, carry):
        sl = pl.ds(i * 16, 16)
        d = d0_v[sl] + d1_v[sl]
        bits = lax.bitcast_convert_type(d, jnp.int32)
        y = lax.bitcast_convert_type(
            magic - lax.shift_right_logical(bits, 1), jnp.float32)
        y = y * (1.5 - 0.5 * d * y * y)
        y = y * (1.5 - 0.5 * d * y * y)
        y = y * (1.5 - 0.5 * d * y * y)
        dis_v[sl] = jnp.where(d > 0.0, y, 0.0)
        return carry

    lax.fori_loop(0, NP1 // 16, dbody, 0)

    def nbody(j, carry):
        def inner(m, carry2):
            sl = pl.ds(m * 16, 16)
            rv = row_v[j, sl]
            cv = col_v[j, sl]
            wv = w_v[j, sl]
            disr = plsc.load_gather(dis_v, [rv])
            disc = plsc.load_gather(dis_v, [cv])
            nv = -disr * wv * disc - jnp.where(rv == cv, 1.0, 0.0)
            norm_v[j, sl] = nv
            return carry2

        return lax.fori_loop(0, CHUNK // 16, inner, carry)

    lax.fori_loop(0, NCH, nbody, 0)
    pltpu.sync_copy(norm_v, norm_hbm.at[wid])


@functools.partial(
    pl.kernel,
    mesh=_mesh,
    compiler_params=pltpu.CompilerParams(needs_layout_passes=False),
    out_type=jax.ShapeDtypeStruct((NC, N, D), jnp.float32),
    scratch_types=[
        pltpu.VMEM((3, CHUNK), jnp.int32),
        pltpu.VMEM((3, CHUNK), jnp.int32),
        pltpu.VMEM((3, CHUNK), jnp.float32),
        pltpu.VMEM((CHUNK, D), jnp.float32),
        pltpu.VMEM((CHUNK, D), jnp.float32),
        pltpu.VMEM((CHUNK, D), jnp.float32),
        pltpu.VMEM_SHARED((N, D), jnp.float32),
        pltpu.SemaphoreType.DMA,
        pltpu.SemaphoreType.DMA,
        pltpu.SemaphoreType.DMA,
        pltpu.SemaphoreType.DMA,
        pltpu.SemaphoreType.DMA,
        pltpu.SemaphoreType.DMA,
        pltpu.SemaphoreType.DMA,
        pltpu.SemaphoreType.DMA,
        pltpu.SemaphoreType.DMA,
        pltpu.SemaphoreType.DMA,
        pltpu.SemaphoreType.DMA,
        pltpu.SemaphoreType.DMA,
        pltpu.SemaphoreType.DMA,
        pltpu.SemaphoreType.DMA,
        pltpu.SemaphoreType.DMA,
    ],
)
def _prop_kernel(z_hbm, row_hbm, col_hbm, norm_hbm, out_hbm,
                 rowr, colr, nrmr, gb0, gb1, gb2, accum,
                 gs0, gs1, gs2, ss0, ss1, ss2,
                 rs0, rs1, rs2, cs0, cs1, cs2, ns0, ns1, ns2):
    cid = lax.axis_index("c")
    sid = lax.axis_index("s")
    wid = cid * NS + sid
    gbufs = (gb0, gb1, gb2)
    gsems = (gs0, gs1, gs2)
    ssems = (ss0, ss1, ss2)
    rsems = (rs0, rs1, rs2)
    csems = (cs0, cs1, cs2)
    nsems = (ns0, ns1, ns2)

    def zb(r, carry):
        for m in range(D // 16):
            gb0[r, pl.ds(m * 16, 16)] = _zero16()
        return carry

    lax.fori_loop(0, CHUNK, zb, 0)
    _strip_copies(sid, lambda off, sz: pltpu.sync_copy(
        gb0.at[pl.ds(0, sz)], accum.at[pl.ds(off, sz)]))
    plsc.subcore_barrier()

    def row_dma(j, b):
        return pltpu.make_async_copy(
            row_hbm.at[pl.ds(wid * EPW + j * CHUNK, CHUNK)], rowr.at[b],
            rsems[b])

    def col_dma(j, b):
        return pltpu.make_async_copy(
            col_hbm.at[pl.ds(wid * EPW + j * CHUNK, CHUNK)], colr.at[b],
            csems[b])

    def nrm_dma(j, b):
        return pltpu.make_async_copy(
            norm_hbm.at[pl.ds(wid * EPW + j * CHUNK, CHUNK)], nrmr.at[b],
            nsems[b])

    def gather_dma(b):
        return pltpu.make_async_copy(z_hbm.at[rowr.at[b]], gbufs[b],
                                     gsems[b])

    def scatter_dma(b):
        return pltpu.make_async_copy(gbufs[b], accum.at[colr.at[b]],
                                     ssems[b])

    def scale(b, buf):
        @plsc.parallel_loop(0, CHUNK, unroll=4)
        def _(r):
            nv = plsc.load_gather(nrmr.at[b],
                                  [jnp.full((16,), r, jnp.int32)])
            for m in range(D // 16):
                sl = pl.ds(m * 16, 16)
                buf[r, sl] = buf[r, sl] * nv

    for jj in range(3):
        row_dma(jj, jj).start()
        nrm_dma(jj, jj).start()
    for jj in range(2):
        col_dma(jj, jj).start()
    for b in range(2):
        row_dma(b, b).wait()
        gather_dma(b).start()

    def triple(t, carry):
        for p in range(3):
            j = 3 * t + p
            ba, bc = p % 3, (p + 2) % 3
            last = NCH // 3 - 1

            def prefetch():
                row_dma(j + 2, bc).wait()
                gather_dma(bc).start()
                col_dma(j + 2, bc).start()

            if p == 0:
                @pl.when(t > 0)
                def _w():
                    scatter_dma(bc).wait()

                prefetch()
            else:
                @pl.when(t < last)
                def _n():
                    scatter_dma(bc).wait()
                    prefetch()

                @pl.when(t == last)
                def _l():
                    scatter_dma(bc).wait()

            pltpu.make_async_copy(z_hbm.at[rowr.at[ba]], gbufs[ba],
                                  gsems[ba]).wait()

            @pl.when(t < last)
            def _rnext():
                row_dma(j + 3, ba).start()

            nrm_dma(j, ba).wait()
            scale(ba, gbufs[ba])
            col_dma(j, ba).wait()
            pltpu.async_copy(gbufs[ba], accum.at[colr.at[ba]], ssems[ba],
                             add=True)

            @pl.when(t < last)
            def _nnext():
                nrm_dma(j + 3, ba).start()
        return carry

    lax.fori_loop(0, NCH // 3, triple, 0)
    scatter_dma((NCH - 1) % 3).wait()
    plsc.subcore_barrier()
    _strip_copies(sid, lambda off, sz: pltpu.sync_copy(
        accum.at[pl.ds(off, sz)], out_hbm.at[cid, pl.ds(off, sz)]))


def _comb1_body(px_ref, ph_ref, tx_ref, th_ref):
    tx_ref[...] = px_ref[0] + px_ref[1]
    th_ref[...] = ph_ref[0] + ph_ref[1]


def _comb2_body(px_ref, ph_ref, xp_ref, hp_ref, tx_ref, th_ref):
    tx_ref[...] = 2.0 * (px_ref[0] + px_ref[1]) - xp_ref[...]
    th_ref[...] = 2.0 * (ph_ref[0] + ph_ref[1]) - hp_ref[...]


def _pblk():
    return pl.BlockSpec((NC, ROWS_BLK, D), lambda i: (0, i, 0))


def _blk():
    return pl.BlockSpec((ROWS_BLK, D), lambda i: (i, 0))


def _comb1(px, ph):
    return pl.pallas_call(
        _comb1_body,
        grid=(N // ROWS_BLK,),
        in_specs=[_pblk(), _pblk()],
        out_specs=[_blk(), _blk()],
        out_shape=[jax.ShapeDtypeStruct((N, D), jnp.float32)] * 2,
    )(px, ph)


def _comb2(px, ph, xp, hp):
    return pl.pallas_call(
        _comb2_body,
        grid=(N // ROWS_BLK,),
        in_specs=[_pblk(), _pblk(), _blk(), _blk()],
        out_specs=[_blk(), _blk()],
        out_shape=[jax.ShapeDtypeStruct((N, D), jnp.float32)] * 2,
    )(px, ph, xp, hp)


def _gate_kernel(x_ref, tx1_ref, tx2_ref, px3_ref,
                 h_ref, th1_ref, th2_ref, ph3_ref,
                 c_ref, wx_ref, wh_ref, wc_ref, bias_ref,
                 lin_w_ref, lin_b_ref,
                 out_ref, h0_ref, cn_ref):
    f32 = jnp.float32

    def dot(a, b):
        return jnp.dot(a, b, preferred_element_type=f32)

    tx3 = 2.0 * (px3_ref[0] + px3_ref[1]) - tx1_ref[...]
    th3 = 2.0 * (ph3_ref[0] + ph3_ref[1]) - th1_ref[...]
    pre = (dot(x_ref[...], wx_ref[0:128])
           + dot(tx1_ref[...], wx_ref[128:256])
           + dot(tx2_ref[...], wx_ref[256:384])
           + dot(tx3, wx_ref[384:512])
           + dot(h_ref[...], wh_ref[0:128])
           + dot(th1_ref[...], wh_ref[128:256])
           + dot(th2_ref[...], wh_ref[256:384])
           + dot(th3, wh_ref[384:512])
           + bias_ref[0:1, :])
    c = c_ref[...]
    ig = jax.nn.sigmoid(pre[:, 0:128] + wc_ref[0:1, :] * c)
    fg = jax.nn.sigmoid(pre[:, 128:256] + wc_ref[1:2, :] * c)
    tg = jnp.tanh(pre[:, 256:384])
    cn = fg * c + ig * tg
    og = jax.nn.sigmoid(pre[:, 384:512] + wc_ref[2:3, :] * cn)
    h0 = og * jnp.tanh(cn)
    out = dot(jax.nn.relu(h0), lin_w_ref[...]) + lin_b_ref[0:1, :]
    out_ref[...] = out
    h0_ref[...] = h0
    cn_ref[...] = cn


def _gates(x, tx1, tx2, px3, h, th1, th2, ph3, c, wx, wh, wc, bias, lin_w, lin_b):
    full = lambda shp: pl.BlockSpec(shp, lambda i: (0,) * len(shp))
    return pl.pallas_call(
        _gate_kernel,
        grid=(N // ROWS_BLK,),
        in_specs=[
            _blk(), _blk(), _blk(), _pblk(),
            _blk(), _blk(), _blk(), _pblk(),
            _blk(),
            full((512, 512)), full((512, 512)),
            full((8, D)), full((8, 512)),
            full((D, D)), full((8, D)),
        ],
        out_specs=[_blk(), _blk(), _blk()],
        out_shape=[jax.ShapeDtypeStruct((N, D), jnp.float32)] * 3,
    )(x, tx1, tx2, px3, h, th1, th2, ph3, c, wx, wh, wc, bias, lin_w, lin_b)


def kernel(x, edge_index, edge_weight, h, c, params):
    p = params
    row = edge_index[0]
    col = edge_index[1]

    pad = EPAD - E
    apad = jnp.arange(pad, dtype=jnp.int32)
    prow = (2 * apad) % N
    pcol = (2 * apad + 1) % N
    row_p = jnp.concatenate([row, prow]).reshape(NW, NCH, CHUNK)
    col_p = jnp.concatenate([col, pcol]).reshape(NW, NCH, CHUNK)
    w_p = jnp.concatenate(
        [edge_weight, jnp.zeros((pad,), jnp.float32)]).reshape(NW, NCH, CHUNK)

    degp = _deg_kernel(row_p, w_p)
    norm = _norm_kernel(degp, row_p, col_p, w_p)

    row_f = row_p.reshape(NW * EPW)
    col_f = col_p.reshape(NW * EPW)
    norm_f = norm.reshape(NW * EPW)
    px1 = _prop_kernel(x, row_f, col_f, norm_f)
    ph1 = _prop_kernel(h, row_f, col_f, norm_f)
    tx1, th1 = _comb1(px1, ph1)
    px2 = _prop_kernel(tx1, row_f, col_f, norm_f)
    ph2 = _prop_kernel(th1, row_f, col_f, norm_f)
    tx2, th2 = _comb2(px2, ph2, x, h)
    px3 = _prop_kernel(tx2, row_f, col_f, norm_f)
    ph3 = _prop_kernel(th2, row_f, col_f, norm_f)

    gates = ["i", "f", "c", "o"]
    wx = jnp.concatenate([p["Wx_" + g] for g in gates], axis=2).reshape(K * D, 4 * D)
    wh = jnp.concatenate([p["Wh_" + g] for g in gates], axis=2).reshape(K * D, 4 * D)
    wc = jnp.concatenate(
        [p["w_c_i"], p["w_c_f"], p["w_c_o"], jnp.zeros((5, D), jnp.float32)], axis=0)
    bias = jnp.concatenate(
        [(p["bx_" + g] + p["bh_" + g])[None, :] + p["b_" + g] for g in gates], axis=1)
    bias = jnp.concatenate([bias, jnp.zeros((7, 4 * D), jnp.float32)], axis=0)
    lin_b = jnp.concatenate([p["lin_b"][None, :], jnp.zeros((7, D), jnp.float32)], axis=0)

    out, h0, cn = _gates(x, tx1, tx2, px3, h, th1, th2, ph3, c,
                         wx, wh, wc, bias, p["lin_W"], lin_b)
    return (out, h0, cn)


# speedup vs baseline: 1.0115x; 1.0052x over previous
import functools

import jax
import jax.numpy as jnp
from jax import lax
from jax.experimental import pallas as pl
from jax.experimental.pallas import tpu as pltpu
from jax.experimental.pallas import tpu_sc as plsc

N = 10000
NP1 = 10240
E = 320000
D = 128
K = 4
NC, NS = 2, 16
NW = NC * NS
CHUNK = 128
NCH = 81
EPW = CHUNK * NCH
EPAD = EPW * NW

ROWS_BLK = 1000

_mesh = plsc.VectorSubcoreMesh(core_axis_name="c", subcore_axis_name="s")


def _strip_copies(sid, fn):
    base = sid * 632
    for off in (0, 128, 256, 384):
        fn(base + off, 128)

    @pl.when(sid < NS - 1)
    def _full():
        fn(base + 512, 120)

    @pl.when(sid == NS - 1)
    def _last():
        fn(base + 512, 8)


def _zero16():
    return jnp.zeros((16,), jnp.float32)


@functools.partial(
    pl.kernel,
    mesh=_mesh,
    compiler_params=pltpu.CompilerParams(needs_layout_passes=False),
    out_type=jax.ShapeDtypeStruct((NC, NP1), jnp.float32),
    scratch_types=[
        pltpu.VMEM((NCH, CHUNK), jnp.int32),
        pltpu.VMEM((NCH, CHUNK), jnp.float32),
        pltpu.VMEM((NP1 // NS,), jnp.float32),
        pltpu.VMEM_SHARED((NP1,), jnp.float32),
        pltpu.SemaphoreType.DMA,
    ],
)
def _deg_kernel(row_hbm, w_hbm, out_hbm, row_v, w_v, zbuf, accum, dsem):
    cid = lax.axis_index("c")
    sid = lax.axis_index("s")
    wid = cid * NS + sid
    strip = NP1 // NS

    def zb(i, carry):
        zbuf[pl.ds(i * 16, 16)] = _zero16()
        return carry

    lax.fori_loop(0, strip // 16, zb, 0)
    pltpu.sync_copy(zbuf, accum.at[pl.ds(sid * strip, strip)])
    pltpu.sync_copy(row_hbm.at[wid], row_v)
    pltpu.sync_copy(w_hbm.at[wid], w_v)
    plsc.subcore_barrier()

    def body(j, carry):
        pltpu.async_copy(w_v.at[j], accum.at[row_v.at[j]], dsem, add=True)

        @pl.when(j >= 4)
        def _d():
            pltpu.make_async_copy(w_v.at[j - 4],
                                  accum.at[row_v.at[j - 4]], dsem).wait()

        return carry

    lax.fori_loop(0, NCH, body, 0)

    def drain(j, carry):
        pltpu.make_async_copy(w_v.at[j], accum.at[row_v.at[j]], dsem).wait()
        return carry

    lax.fori_loop(NCH - 4, NCH, drain, 0)
    plsc.subcore_barrier()
    pltpu.sync_copy(accum.at[pl.ds(sid * strip, strip)],
                    out_hbm.at[cid, pl.ds(sid * strip, strip)])


@functools.partial(
    pl.kernel,
    mesh=_mesh,
    compiler_params=pltpu.CompilerParams(needs_layout_passes=False),
    out_type=jax.ShapeDtypeStruct((NW, NCH, CHUNK), jnp.float32),
    scratch_types=[
        pltpu.VMEM((NP1,), jnp.float32),
        pltpu.VMEM((NP1,), jnp.float32),
        pltpu.VMEM((NP1,), jnp.float32),
        pltpu.VMEM((NCH, CHUNK), jnp.int32),
        pltpu.VMEM((NCH, CHUNK), jnp.int32),
        pltpu.VMEM((NCH, CHUNK), jnp.float32),
        pltpu.VMEM((NCH, CHUNK), jnp.float32),
    ],
)
def _norm_kernel(degp_hbm, row_hbm, col_hbm, w_hbm, norm_hbm,
                 d0_v, d1_v, dis_v, row_v, col_v, w_v, norm_v):
    cid = lax.axis_index("c")
    sid = lax.axis_index("s")
    wid = cid * NS + sid
    pltpu.sync_copy(degp_hbm.at[0], d0_v)
    pltpu.sync_copy(degp_hbm.at[1], d1_v)
    pltpu.sync_copy(row_hbm.at[wid], row_v)
    pltpu.sync_copy(col_hbm.at[wid], col_v)
    pltpu.sync_copy(w_hbm.at[wid], w_v)

    magic = jnp.full((16,), 0x5F3759DF, jnp.int32)

    def dbody(i, carry):
        sl = pl.ds(i * 16, 16)
        d = d0_v[sl] + d1_v[sl]
        bits = lax.bitcast_convert_type(d, jnp.int32)
        y = lax.bitcast_convert_type(
            magic - lax.shift_right_logical(bits, 1), jnp.float32)
        y = y * (1.5 - 0.5 * d * y * y)
        y = y * (1.5 - 0.5 * d * y * y)
        y = y * (1.5 - 0.5 * d * y * y)
        dis_v[sl] = jnp.where(d > 0.0, y, 0.0)
        return carry

    lax.fori_loop(0, NP1 // 16, dbody, 0)

    def nbody(j, carry):
        def inner(m, carry2):
            sl = pl.ds(m * 16, 16)
            rv = row_v[j, sl]
            cv = col_v[j, sl]
            wv = w_v[j, sl]
            disr = plsc.load_gather(dis_v, [rv])
            disc = plsc.load_gather(dis_v, [cv])
            nv = -disr * wv * disc - jnp.where(rv == cv, 1.0, 0.0)
            norm_v[j, sl] = nv
            return carry2

        return lax.fori_loop(0, CHUNK // 16, inner, carry)

    lax.fori_loop(0, NCH, nbody, 0)
    pltpu.sync_copy(norm_v, norm_hbm.at[wid])


@functools.partial(
    pl.kernel,
    mesh=_mesh,
    compiler_params=pltpu.CompilerParams(needs_layout_passes=False),
    out_type=jax.ShapeDtypeStruct((NC, N, D), jnp.float32),
    scratch_types=[
        pltpu.VMEM((3, CHUNK), jnp.int32),
        pltpu.VMEM((3, CHUNK), jnp.int32),
        pltpu.VMEM((3, CHUNK), jnp.float32),
        pltpu.VMEM((CHUNK, D), jnp.float32),
        pltpu.VMEM((CHUNK, D), jnp.float32),
        pltpu.VMEM((CHUNK, D), jnp.float32),
        pltpu.VMEM_SHARED((N, D), jnp.float32),
        pltpu.SemaphoreType.DMA,
        pltpu.SemaphoreType.DMA,
        pltpu.SemaphoreType.DMA,
        pltpu.SemaphoreType.DMA,
        pltpu.SemaphoreType.DMA,
        pltpu.SemaphoreType.DMA,
        pltpu.SemaphoreType.DMA,
        pltpu.SemaphoreType.DMA,
        pltpu.SemaphoreType.DMA,
        pltpu.SemaphoreType.DMA,
        pltpu.SemaphoreType.DMA,
        pltpu.SemaphoreType.DMA,
        pltpu.SemaphoreType.DMA,
        pltpu.SemaphoreType.DMA,
        pltpu.SemaphoreType.DMA,
    ],
)
def _prop_kernel(z_hbm, row_hbm, col_hbm, norm_hbm, out_hbm,
                 rowr, colr, nrmr, gb0, gb1, gb2, accum,
                 gs0, gs1, gs2, ss0, ss1, ss2,
                 rs0, rs1, rs2, cs0, cs1, cs2, ns0, ns1, ns2):
    cid = lax.axis_index("c")
    sid = lax.axis_index("s")
    wid = cid * NS + sid
    gbufs = (gb0, gb1, gb2)
    gsems = (gs0, gs1, gs2)
    ssems = (ss0, ss1, ss2)
    rsems = (rs0, rs1, rs2)
    csems = (cs0, cs1, cs2)
    nsems = (ns0, ns1, ns2)

    def zb(r, carry):
        for m in range(D // 16):
            gb0[r, pl.ds(m * 16, 16)] = _zero16()
        return carry

    lax.fori_loop(0, CHUNK, zb, 0)
    _strip_copies(sid, lambda off, sz: pltpu.sync_copy(
        gb0.at[pl.ds(0, sz)], accum.at[pl.ds(off, sz)]))
    plsc.subcore_barrier()

    def row_dma(j, b):
        return pltpu.make_async_copy(
            row_hbm.at[pl.ds(wid * EPW + j * CHUNK, CHUNK)], rowr.at[b],
            rsems[b])

    def col_dma(j, b):
        return pltpu.make_async_copy(
            col_hbm.at[pl.ds(wid * EPW + j * CHUNK, CHUNK)], colr.at[b],
            csems[b])

    def nrm_dma(j, b):
        return pltpu.make_async_copy(
            norm_hbm.at[pl.ds(wid * EPW + j * CHUNK, CHUNK)], nrmr.at[b],
            nsems[b])

    def gather_dma(b):
        return pltpu.make_async_copy(z_hbm.at[rowr.at[b]], gbufs[b],
                                     gsems[b])

    def scatter_dma(b):
        return pltpu.make_async_copy(gbufs[b], accum.at[colr.at[b]],
                                     ssems[b])

    def scale(b, buf):
        @plsc.parallel_loop(0, CHUNK, unroll=4)
        def _(r):
            nv = plsc.load_gather(nrmr.at[b],
                                  [jnp.full((16,), r, jnp.int32)])
            for m in range(D // 16):
                sl = pl.ds(m * 16, 16)
                buf[r, sl] = buf[r, sl] * nv

    for jj in range(3):
        row_dma(jj, jj).start()
        nrm_dma(jj, jj).start()
    for jj in range(2):
        col_dma(jj, jj).start()
    for b in range(2):
        row_dma(b, b).wait()
        gather_dma(b).start()

    def triple(t, carry):
        for p in range(3):
            j = 3 * t + p
            ba, bc = p % 3, (p + 2) % 3
            last = NCH // 3 - 1

            def prefetch():
                row_dma(j + 2, bc).wait()
                gather_dma(bc).start()
                col_dma(j + 2, bc).start()

            if p == 0:
                @pl.when(t > 0)
                def _w():
                    scatter_dma(bc).wait()

                prefetch()
            else:
                @pl.when(t < last)
                def _n():
                    scatter_dma(bc).wait()
                    prefetch()

                @pl.when(t == last)
                def _l():
                    scatter_dma(bc).wait()

            pltpu.make_async_copy(z_hbm.at[rowr.at[ba]], gbufs[ba],
                                  gsems[ba]).wait()

            @pl.when(t < last)
            def _rnext():
                row_dma(j + 3, ba).start()

            nrm_dma(j, ba).wait()
            scale(ba, gbufs[ba])
            col_dma(j, ba).wait()
            pltpu.async_copy(gbufs[ba], accum.at[colr.at[ba]], ssems[ba],
                             add=True)

            @pl.when(t < last)
            def _nnext():
                nrm_dma(j + 3, ba).start()
        return carry

    lax.fori_loop(0, NCH // 3, triple, 0)
    scatter_dma((NCH - 1) % 3).wait()
    plsc.subcore_barrier()
    _strip_copies(sid, lambda off, sz: pltpu.sync_copy(
        accum.at[pl.ds(off, sz)], out_hbm.at[cid, pl.ds(off, sz)]))


def _comb1_body(px_ref, ph_ref, tx_ref, th_ref):
    tx_ref[...] = px_ref[0] + px_ref[1]
    th_ref[...] = ph_ref[0] + ph_ref[1]


def _comb2_body(px_ref, ph_ref, xp_ref, hp_ref, tx_ref, th_ref):
    tx_ref[...] = 2.0 * (px_ref[0] + px_ref[1]) - xp_ref[...]
    th_ref[...] = 2.0 * (ph_ref[0] + ph_ref[1]) - hp_ref[...]


def _pblk():
    return pl.BlockSpec((NC, ROWS_BLK, D), lambda i: (0, i, 0))


def _blk():
    return pl.BlockSpec((ROWS_BLK, D), lambda i: (i, 0))


def _comb1(px, ph):
    return pl.pallas_call(
        _comb1_body,
        grid=(N // ROWS_BLK,),
        in_specs=[_pblk(), _pblk()],
        out_specs=[_blk(), _blk()],
        out_shape=[jax.ShapeDtypeStruct((N, D), jnp.float32)] * 2,
    )(px, ph)


def _comb2(px, ph, xp, hp):
    return pl.pallas_call(
        _comb2_body,
        grid=(N // ROWS_BLK,),
        in_specs=[_pblk(), _pblk(), _blk(), _blk()],
        out_specs=[_blk(), _blk()],
        out_shape=[jax.ShapeDtypeStruct((N, D), jnp.float32)] * 2,
    )(px, ph, xp, hp)


def _gate_kernel(x_ref, tx1_ref, tx2_ref, px3_ref,
                 h_ref, th1_ref, th2_ref, ph3_ref,
                 c_ref, wx_ref, wh_ref, wc_ref, bias_ref,
                 lin_w_ref, lin_b_ref,
                 out_ref, h0_ref, cn_ref):
    f32 = jnp.float32

    def dot(a, b):
        return jnp.dot(a, b, preferred_element_type=f32)

    tx3 = 2.0 * (px3_ref[0] + px3_ref[1]) - tx1_ref[...]
    th3 = 2.0 * (ph3_ref[0] + ph3_ref[1]) - th1_ref[...]
    pre = (dot(x_ref[...], wx_ref[0:128])
           + dot(tx1_ref[...], wx_ref[128:256])
           + dot(tx2_ref[...], wx_ref[256:384])
           + dot(tx3, wx_ref[384:512])
           + dot(h_ref[...], wh_ref[0:128])
           + dot(th1_ref[...], wh_ref[128:256])
           + dot(th2_ref[...], wh_ref[256:384])
           + dot(th3, wh_ref[384:512])
           + bias_ref[0:1, :])
    c = c_ref[...]
    ig = jax.nn.sigmoid(pre[:, 0:128] + wc_ref[0:1, :] * c)
    fg = jax.nn.sigmoid(pre[:, 128:256] + wc_ref[1:2, :] * c)
    tg = jnp.tanh(pre[:, 256:384])
    cn = fg * c + ig * tg
    og = jax.nn.sigmoid(pre[:, 384:512] + wc_ref[2:3, :] * cn)
    h0 = og * jnp.tanh(cn)
    out = dot(jax.nn.relu(h0), lin_w_ref[...]) + lin_b_ref[0:1, :]
    out_ref[...] = out
    h0_ref[...] = h0
    cn_ref[...] = cn


def _gates(x, tx1, tx2, px3, h, th1, th2, ph3, c, wx, wh, wc, bias, lin_w, lin_b):
    full = lambda shp: pl.BlockSpec(shp, lambda i: (0,) * len(shp))
    return pl.pallas_call(
        _gate_kernel,
        grid=(N // ROWS_BLK,),
        in_specs=[
            _blk(), _blk(), _blk(), _pblk(),
            _blk(), _blk(), _blk(), _pblk(),
            _blk(),
            full((512, 512)), full((512, 512)),
            full((8, D)), full((8, 512)),
            full((D, D)), full((8, D)),
        ],
        out_specs=[_blk(), _blk(), _blk()],
        out_shape=[jax.ShapeDtypeStruct((N, D), jnp.float32)] * 3,
    )(x, tx1, tx2, px3, h, th1, th2, ph3, c, wx, wh, wc, bias, lin_w, lin_b)


def kernel(x, edge_index, edge_weight, h, c, params):
    p = params
    row = edge_index[0]
    col = edge_index[1]

    pad = EPAD - E
    apad = jnp.arange(pad, dtype=jnp.int32)
    prow = (2 * apad) % N
    pcol = (2 * apad + 1) % N
    row_p = jnp.concatenate([row, prow]).reshape(NW, NCH, CHUNK)
    col_p = jnp.concatenate([col, pcol]).reshape(NW, NCH, CHUNK)
    w_p = jnp.concatenate(
        [edge_weight, jnp.zeros((pad,), jnp.float32)]).reshape(NW, NCH, CHUNK)

    degp = _deg_kernel(row_p, w_p)
    norm = _norm_kernel(degp, row_p, col_p, w_p)

    row_f = row_p.reshape(NW * EPW)
    col_f = col_p.reshape(NW * EPW)
    norm_f = norm.reshape(NW * EPW)
    px1 = _prop_kernel(x, row_f, col_f, norm_f)
    ph1 = _prop_kernel(h, row_f, col_f, norm_f)
    tx1, th1 = _comb1(px1, ph1)
    px2 = _prop_kernel(tx1, row_f, col_f, norm_f)
    ph2 = _prop_kernel(th1, row_f, col_f, norm_f)
    tx2, th2 = _comb2(px2, ph2, x, h)
    px3 = _prop_kernel(tx2, row_f, col_f, norm_f)
    ph3 = _prop_kernel(th2, row_f, col_f, norm_f)

    gates = ["i", "f", "c", "o"]
    wx = jnp.concatenate([p["Wx_" + g] for g in gates], axis=2).reshape(K * D, 4 * D)
    wh = jnp.concatenate([p["Wh_" + g] for g in gates], axis=2).reshape(K * D, 4 * D)
    wc = jnp.concatenate(
        [p["w_c_i"], p["w_c_f"], p["w_c_o"], jnp.zeros((5, D), jnp.float32)], axis=0)
    bias = jnp.concatenate(
        [(p["bx_" + g] + p["bh_" + g])[None, :] + p["b_" + g] for g in gates], axis=1)
    bias = jnp.concatenate([bias, jnp.zeros((7, 4 * D), jnp.float32)], axis=0)
    lin_b = jnp.concatenate([p["lin_b"][None, :], jnp.zeros((7, D), jnp.float32)], axis=0)

    out, h0, cn = _gates(x, tx1, tx2, px3, h, th1, th2, ph3, c,
                         wx, wh, wc, bias, p["lin_W"], lin_b)
    return (out, h0, cn)
